# baseline (speedup 1.0000x reference)
import jax
import jax.numpy as jnp
from jax.experimental import pallas as pl

_CBLK = 128
_S = 4096


def _body(x_ref, gt_ref, w_ref, xo_ref, o_ref):
    K = w_ref.shape[1]
    HW = x_ref.shape[2]
    f32 = jnp.float32
    nch = HW // _S
    hp = jax.lax.Precision.HIGHEST

    def onehot(i):
        gt_s = gt_ref[0, :, pl.ds(i * _S, _S)]
        cls = jax.lax.broadcasted_iota(jnp.int32, (K, _S), 0)
        return (cls == gt_s).astype(f32)

    fsum = jnp.zeros((K, _CBLK), f32)
    fsq = jnp.zeros((K, _CBLK), f32)
    cnt = jnp.zeros((K, 1), f32)
    for i in range(nch):
        oh = onehot(i)
        xc = x_ref[0, :, pl.ds(i * _S, _S)]
        fsum = fsum + jax.lax.dot_general(
            oh, xc, (((1,), (1,)), ((), ())), preferred_element_type=f32)
        fsq = fsq + jax.lax.dot_general(
            oh, xc * xc, (((1,), (1,)), ((), ())), preferred_element_type=f32)
        cnt = cnt + jnp.sum(oh, axis=1, keepdims=True)

    rc = 1.0 / jnp.where(cnt > 0, cnt, 1.0)
    mean = fsum * rc
    var = jnp.maximum(fsq * rc - mean * mean, 0.0)
    std = jnp.sqrt(var) + 1e-7
    wm = w_ref[0]
    sm = jax.lax.dot_general(
        wm, mean, (((1,), (0,)), ((), ())), precision=hp,
        preferred_element_type=f32)
    ss = jax.lax.dot_general(
        wm, std, (((1,), (0,)), ((), ())), precision=hp,
        preferred_element_type=f32)
    rss = ss / std

    def gather(tbl, oh):
        return jax.lax.dot_general(
            tbl, oh, (((0,), (0,)), ((), ())),
            preferred_element_type=f32)

    for i in range(nch):
        oh = onehot(i)
        xc = x_ref[0, :, pl.ds(i * _S, _S)]
        mg = gather(mean, oh)
        rg = gather(rss, oh)
        sg = gather(sm, oh)
        xo_ref[0, :, pl.ds(i * _S, _S)] = xc
        o_ref[0, :, pl.ds(i * _S, _S)] = (xc - mg) * rg + sg


def kernel(x, gt, aug_rand_info):
    B, C, H, W = x.shape
    K = aug_rand_info.shape[1]
    HW = H * W
    xf = x.reshape(B, C, HW)
    gtf = gt.reshape(B, 1, HW).astype(jnp.int32)
    w = aug_rand_info.reshape(B, K, K)
    nc = C // _CBLK
    xs = pl.pallas_call(
        _body,
        grid=(B, nc),
        in_specs=[
            pl.BlockSpec((1, _CBLK, HW), lambda b, c: (b, c, 0)),
            pl.BlockSpec((1, 1, HW), lambda b, c: (b, 0, 0)),
            pl.BlockSpec((1, K, K), lambda b, c: (b, 0, 0)),
        ],
        out_specs=[
            pl.BlockSpec((1, _CBLK, HW), lambda b, c: (b, c, 0)),
            pl.BlockSpec((1, _CBLK, HW), lambda b, c: (b, c, 0)),
        ],
        out_shape=[
            jax.ShapeDtypeStruct((B, C, HW), x.dtype),
            jax.ShapeDtypeStruct((B, C, HW), x.dtype),
        ],
    )(xf, gtf, w)
    return (xs[0].reshape(B, C, H, W), xs[1].reshape(B, C, H, W))


# speedup vs baseline: 1.0799x; 1.0799x over previous
import jax
import jax.numpy as jnp
from jax.experimental import pallas as pl
from jax.experimental.pallas import tpu as pltpu

_S = 2048
_NIN = 4
_f32 = jnp.float32


def _body(x_hbm, gt_ref, w_ref, o_hbm, xbuf, in_sems, out_sems):
    b = pl.program_id(0)
    B = pl.num_programs(0)
    C = x_hbm.shape[1]
    HW = x_hbm.shape[2]
    K = w_ref.shape[1]
    nout = HW // _S
    cin = C // _NIN
    p = jax.lax.rem(b, 2)
    pn = jax.lax.rem(b + 1, 2)

    def in_copy(batch, parity, j):
        return pltpu.make_async_copy(
            x_hbm.at[batch, pl.ds(j * cin, cin), :],
            xbuf.at[parity, pl.ds(j * cin, cin), :],
            in_sems.at[parity, j])

    def out_copy(batch, parity, i):
        return pltpu.make_async_copy(
            xbuf.at[parity, :, pl.ds(i * _S, _S)],
            o_hbm.at[batch, :, pl.ds(i * _S, _S)],
            out_sems.at[parity, i])

    @pl.when(b == 0)
    def _():
        for j in range(_NIN):
            in_copy(0, 0, j).start()

    for j in range(_NIN):
        in_copy(b, p, j).wait()

    @pl.when(b > 0)
    def _():
        for i in range(nout):
            out_copy(b - 1, pn, i).wait()

    @pl.when(b + 1 < B)
    def _():
        for j in range(_NIN):
            in_copy(b + 1, pn, j).start()

    def onehot(i):
        gt_s = gt_ref[b, :, pl.ds(i * _S, _S)]
        cls = jax.lax.broadcasted_iota(jnp.int32, (K, _S), 0)
        return (cls == gt_s).astype(_f32)

    fsum = jnp.zeros((K, C), _f32)
    fsq = jnp.zeros((K, C), _f32)
    cnt = jnp.zeros((K, 1), _f32)
    for i in range(nout):
        oh = onehot(i)
        xc = xbuf[p, :, pl.ds(i * _S, _S)]
        fsum = fsum + jax.lax.dot_general(
            oh, xc, (((1,), (1,)), ((), ())), preferred_element_type=_f32)
        fsq = fsq + jax.lax.dot_general(
            oh, xc * xc, (((1,), (1,)), ((), ())), preferred_element_type=_f32)
        cnt = cnt + jnp.sum(oh, axis=1, keepdims=True)

    rc = 1.0 / jnp.where(cnt > 0, cnt, 1.0)
    mean = fsum * rc
    var = jnp.maximum(fsq * rc - mean * mean, 0.0)
    std = jnp.sqrt(var) + 1e-7
    wm = w_ref[b]
    hp = jax.lax.Precision.HIGHEST
    sm = jax.lax.dot_general(
        wm, mean, (((1,), (0,)), ((), ())), precision=hp,
        preferred_element_type=_f32)
    ss = jax.lax.dot_general(
        wm, std, (((1,), (0,)), ((), ())), precision=hp,
        preferred_element_type=_f32)
    rss = ss / std

    def gather(tbl, oh):
        return jax.lax.dot_general(
            tbl, oh, (((0,), (0,)), ((), ())),
            preferred_element_type=_f32)

    for i in range(nout):
        oh = onehot(i)
        xc = xbuf[p, :, pl.ds(i * _S, _S)]
        mg = gather(mean, oh)
        rg = gather(rss, oh)
        sg = gather(sm, oh)
        xbuf[p, :, pl.ds(i * _S, _S)] = (xc - mg) * rg + sg
        out_copy(b, p, i).start()

    @pl.when(b == B - 1)
    def _():
        for i in range(nout):
            out_copy(b, p, i).wait()


def kernel(x, gt, aug_rand_info):
    B, C, H, W = x.shape
    K = aug_rand_info.shape[1]
    HW = H * W
    xf = x.reshape(B, C, HW)
    gtf = gt.reshape(B, 1, HW).astype(jnp.int32)
    w = aug_rand_info.reshape(B, K, K)
    xs = pl.pallas_call(
        _body,
        grid=(B,),
        in_specs=[
            pl.BlockSpec(memory_space=pl.ANY),
            pl.BlockSpec(memory_space=pltpu.MemorySpace.VMEM),
            pl.BlockSpec(memory_space=pltpu.MemorySpace.VMEM),
        ],
        out_specs=pl.BlockSpec(memory_space=pl.ANY),
        out_shape=jax.ShapeDtypeStruct((B, C, HW), x.dtype),
        scratch_shapes=[
            pltpu.VMEM((2, C, HW), _f32),
            pltpu.SemaphoreType.DMA((2, _NIN)),
            pltpu.SemaphoreType.DMA((2, HW // _S)),
        ],
    )(xf, gtf, w)
    return (x, xs.reshape(B, C, H, W))


# speedup vs baseline: 1.1023x; 1.0207x over previous
import jax
import jax.numpy as jnp
from jax.experimental import pallas as pl
from jax.experimental.pallas import tpu as pltpu

_S = 2048
_NIN = 4
_NBUF = 3
_f32 = jnp.float32


def _body(x_hbm, gt_ref, w_ref, o_hbm, xbuf, in_sems, out_sems):
    b = pl.program_id(0)
    B = pl.num_programs(0)
    C = x_hbm.shape[1]
    HW = x_hbm.shape[2]
    K = w_ref.shape[1]
    nout = HW // _S
    cin = C // _NIN
    p = jax.lax.rem(b, _NBUF)
    pn = jax.lax.rem(b + 1, _NBUF)

    def in_copy(batch, parity, j):
        return pltpu.make_async_copy(
            x_hbm.at[batch, pl.ds(j * cin, cin), :],
            xbuf.at[parity, pl.ds(j * cin, cin), :],
            in_sems.at[parity, j])

    def out_copy(batch, parity, i):
        return pltpu.make_async_copy(
            xbuf.at[parity, :, pl.ds(i * _S, _S)],
            o_hbm.at[batch, :, pl.ds(i * _S, _S)],
            out_sems.at[parity, i])

    @pl.when(b == 0)
    def _():
        for j in range(_NIN):
            in_copy(0, 0, j).start()

    for j in range(_NIN):
        in_copy(b, p, j).wait()

    @pl.when(b >= 2)
    def _():
        for i in range(nout):
            out_copy(b - 2, pn, i).wait()

    @pl.when(b + 1 < B)
    def _():
        for j in range(_NIN):
            in_copy(b + 1, pn, j).start()

    cls = jax.lax.broadcasted_iota(jnp.int32, (K, HW), 0)
    oh_full = (cls == gt_ref[b]).astype(_f32)

    fsum = jnp.zeros((K, C), _f32)
    fsq = jnp.zeros((K, C), _f32)
    for i in range(nout):
        oh = oh_full[:, i * _S:(i + 1) * _S]
        xc = xbuf[p, :, pl.ds(i * _S, _S)]
        fsum = fsum + jax.lax.dot_general(
            oh, xc, (((1,), (1,)), ((), ())), preferred_element_type=_f32)
        fsq = fsq + jax.lax.dot_general(
            oh, xc * xc, (((1,), (1,)), ((), ())), preferred_element_type=_f32)
    cnt = jnp.sum(oh_full, axis=1, keepdims=True)

    rc = 1.0 / jnp.where(cnt > 0, cnt, 1.0)
    mean = fsum * rc
    var = jnp.maximum(fsq * rc - mean * mean, 0.0)
    std = jnp.sqrt(var) + 1e-7
    wm = w_ref[b]
    hp = jax.lax.Precision.HIGHEST
    sm = jax.lax.dot_general(
        wm, mean, (((1,), (0,)), ((), ())), precision=hp,
        preferred_element_type=_f32)
    ss = jax.lax.dot_general(
        wm, std, (((1,), (0,)), ((), ())), precision=hp,
        preferred_element_type=_f32)
    rss = ss / std

    def gather(tbl, oh):
        return jax.lax.dot_general(
            tbl, oh, (((0,), (0,)), ((), ())),
            preferred_element_type=_f32)

    for i in range(nout):
        oh = oh_full[:, i * _S:(i + 1) * _S]
        xc = xbuf[p, :, pl.ds(i * _S, _S)]
        mg = gather(mean, oh)
        rg = gather(rss, oh)
        sg = gather(sm, oh)
        xbuf[p, :, pl.ds(i * _S, _S)] = (xc - mg) * rg + sg
        out_copy(b, p, i).start()

    @pl.when(b == B - 1)
    def _():
        for i in range(nout):
            out_copy(b - 1, jax.lax.rem(b - 1, _NBUF), i).wait()
        for i in range(nout):
            out_copy(b, p, i).wait()


def kernel(x, gt, aug_rand_info):
    B, C, H, W = x.shape
    K = aug_rand_info.shape[1]
    HW = H * W
    xf = x.reshape(B, C, HW)
    gtf = gt.reshape(B, 1, HW).astype(jnp.int32)
    w = aug_rand_info.reshape(B, K, K)
    xs = pl.pallas_call(
        _body,
        grid=(B,),
        in_specs=[
            pl.BlockSpec(memory_space=pl.ANY),
            pl.BlockSpec(memory_space=pltpu.MemorySpace.VMEM),
            pl.BlockSpec(memory_space=pltpu.MemorySpace.VMEM),
        ],
        out_specs=pl.BlockSpec(memory_space=pl.ANY),
        out_shape=jax.ShapeDtypeStruct((B, C, HW), x.dtype),
        scratch_shapes=[
            pltpu.VMEM((_NBUF, C, HW), _f32),
            pltpu.SemaphoreType.DMA((_NBUF, _NIN)),
            pltpu.SemaphoreType.DMA((_NBUF, HW // _S)),
        ],
    )(xf, gtf, w)
    return (x, xs.reshape(B, C, H, W))
